# baseline (speedup 1.0000x reference)
import functools

import jax
import jax.numpy as jnp
from jax import lax
from jax.experimental import pallas as pl
from jax.experimental.pallas import tpu as pltpu
from jax.experimental.pallas import tpu_sc as plsc

G = 8
K = 2
C = 96
B = 16
HW = 56
P = HW * HW
HIDDEN = 64
LB_COEF = 0.01
NC = 2
NS = 16


def _pool_mlp_body(g0, g1, g2, g3, g4, g5, g6, g7, w1, b1, w2, b2,
                   logits_ref, logits_t_ref, probs_ref, loss_ref, pool_scr):
    b = pl.program_id(0)
    grefs = (g0, g1, g2, g3, g4, g5, g6, g7)
    for g in range(G):
        x = grefs[g][0]
        s1 = jnp.sum(x, axis=1)
        pooled = jnp.sum(s1, axis=-1) * (1.0 / P)
        pool_scr[b, g, :] = pooled

    @pl.when(b == B - 1)
    def _():
        cols = []
        for g in range(G):
            pg = pool_scr[:, g, :]
            h = jnp.maximum(
                jnp.dot(pg, w1[g], preferred_element_type=jnp.float32)
                + b1[g][None, :], 0.0)
            lgt = jnp.dot(h, w2[g], preferred_element_type=jnp.float32) \
                + b2[g][None, :]
            cols.append(lgt)
        logits = jnp.concatenate(cols, axis=1)
        logits_ref[...] = logits
        logits_t_ref[...] = logits.T
        m = jnp.max(logits, axis=1, keepdims=True)
        e = jnp.exp(logits - m)
        probs = e / jnp.sum(e, axis=1, keepdims=True)
        probs_ref[...] = probs
        imp = jnp.mean(probs, axis=0)
        loss_ref[...] = jnp.full((1, 1), LB_COEF * G) * jnp.sum(imp * imp)


def _pool_mlp(groups, w1, b1, w2, b2, interpret=False):
    f32 = jnp.float32
    return pl.pallas_call(
        _pool_mlp_body,
        grid=(B,),
        in_specs=[pl.BlockSpec((1, C, HW, HW), lambda b: (b, 0, 0, 0))
                  for _ in range(G)]
        + [
            pl.BlockSpec((G, C, HIDDEN), lambda b: (0, 0, 0)),
            pl.BlockSpec((G, HIDDEN), lambda b: (0, 0)),
            pl.BlockSpec((G, HIDDEN, 1), lambda b: (0, 0, 0)),
            pl.BlockSpec((G, 1), lambda b: (0, 0)),
        ],
        out_specs=[
            pl.BlockSpec((B, G), lambda b: (0, 0)),
            pl.BlockSpec((G, B), lambda b: (0, 0)),
            pl.BlockSpec((B, G), lambda b: (0, 0)),
            pl.BlockSpec((1, 1), lambda b: (0, 0)),
        ],
        out_shape=[
            jax.ShapeDtypeStruct((B, G), f32),
            jax.ShapeDtypeStruct((G, B), f32),
            jax.ShapeDtypeStruct((B, G), f32),
            jax.ShapeDtypeStruct((1, 1), f32),
        ],
        scratch_shapes=[pltpu.VMEM((B, G, C), f32)],
        interpret=interpret,
    )(*groups, w1, b1, w2, b2)


def _route_body(lg_t_hbm, mask_t_hbm, i1_hbm, i2_hbm,
                lg_v, mk_v, i1_v, i2_v):
    wid = lax.axis_index("s") * NC + lax.axis_index("c")

    pltpu.sync_copy(lg_t_hbm, lg_v)

    neg = jnp.full((16,), -3.0e38, jnp.float32)
    m1 = neg
    i1 = jnp.zeros((16,), jnp.int32)
    for g in range(G):
        v = lg_v[g]
        better = v > m1
        m1 = jnp.where(better, v, m1)
        i1 = jnp.where(better, g, i1)
    m2 = neg
    i2 = jnp.zeros((16,), jnp.int32)
    for g in range(G):
        v = lg_v[g]
        ok = (v > m2) & (i1 != g)
        m2 = jnp.where(ok, v, m2)
        i2 = jnp.where(ok, g, i2)

    for g in range(G):
        sel = (i1 == g) | (i2 == g)
        mk_v[g] = jnp.where(sel, 1.0, 0.0).astype(jnp.float32)
    i1_v[...] = i1
    i2_v[...] = i2

    @pl.when(wid == 0)
    def _():
        pltpu.sync_copy(mk_v, mask_t_hbm)
        pltpu.sync_copy(i1_v, i1_hbm)
        pltpu.sync_copy(i2_v, i2_hbm)


def _route(logits_t):
    f32 = jnp.float32
    run = pl.kernel(
        _route_body,
        out_type=[
            jax.ShapeDtypeStruct((G, B), f32),
            jax.ShapeDtypeStruct((B,), jnp.int32),
            jax.ShapeDtypeStruct((B,), jnp.int32),
        ],
        mesh=plsc.VectorSubcoreMesh(core_axis_name="c", subcore_axis_name="s"),
        compiler_params=pltpu.CompilerParams(needs_layout_passes=False),
        scratch_types=[
            pltpu.VMEM((G, 16), f32),
            pltpu.VMEM((G, 16), f32),
            pltpu.VMEM((16,), jnp.int32),
            pltpu.VMEM((16,), jnp.int32),
        ],
    )
    return run(logits_t)


def _gather_body(idx1_sm, idx2_sm, g0, g1, g2, g3, g4, g5, g6, g7,
                 out_ref, sem):
    grefs = (g0, g1, g2, g3, g4, g5, g6, g7)
    for w in range(B * K):
        b, k = divmod(w, K)
        idx = idx1_sm[b] if k == 0 else idx2_sm[b]
        for g in range(G):
            @pl.when(idx == g)
            def _(g=g, b=b, k=k):
                pltpu.make_async_copy(
                    grefs[g].at[b],
                    out_ref.at[b, pl.ds(k * C, C)],
                    sem,
                ).start()
    for _ in range(B * K):
        pltpu.make_async_copy(
            grefs[0].at[0], out_ref.at[0, pl.ds(0, C)], sem,
        ).wait()


def _gather(idx1, idx2, groups, interpret=False):
    f32 = jnp.float32
    return pl.pallas_call(
        _gather_body,
        in_specs=[pl.BlockSpec(memory_space=pltpu.SMEM),
                  pl.BlockSpec(memory_space=pltpu.SMEM)]
        + [pl.BlockSpec(memory_space=pl.ANY) for _ in range(G)],
        out_specs=pl.BlockSpec(memory_space=pl.ANY),
        out_shape=jax.ShapeDtypeStruct((B, K * C, HW, HW), f32),
        scratch_shapes=[pltpu.SemaphoreType.DMA],
        interpret=interpret,
    )(idx1, idx2, *groups)


def kernel(groups_0, groups_1, groups_2, groups_3, groups_4, groups_5,
           groups_6, groups_7, W1, b1, W2, b2):
    gs = (groups_0, groups_1, groups_2, groups_3, groups_4, groups_5,
          groups_6, groups_7)
    logits, logits_t, soft_probs, loss11 = _pool_mlp(gs, W1, b1, W2, b2)
    mask_t, idx1, idx2 = _route(logits_t)
    out = _gather(idx1, idx2, gs)
    hard_mask = mask_t.T
    load_loss = loss11[0, 0]
    return (out, logits, hard_mask, soft_probs, load_loss)


# speedup vs baseline: 6.1536x; 6.1536x over previous
import functools

import jax
import jax.numpy as jnp
from jax import lax
from jax.experimental import pallas as pl
from jax.experimental.pallas import tpu as pltpu
from jax.experimental.pallas import tpu_sc as plsc

G = 8
K = 2
C = 96
B = 16
HW = 56
P = HW * HW
HIDDEN = 64
LB_COEF = 0.01
NC = 2
NS = 16

RPC = HW
RPB = C * RPC
NCHUNK = 12
CCH = C // NCHUNK
RCH = CCH * RPC


def _pool_mlp_body(g0, g1, g2, g3, g4, g5, g6, g7, w1, b1, w2, b2,
                   logits_ref, logits_t_ref, probs_ref, loss_ref, pool_scr):
    b = pl.program_id(0)
    grefs = (g0, g1, g2, g3, g4, g5, g6, g7)
    for g in range(G):
        x = grefs[g][...].reshape(C, HW, HW)
        s1 = jnp.sum(x, axis=1)
        pooled = jnp.sum(s1, axis=-1) * (1.0 / P)
        pool_scr[b, g, :] = pooled

    @pl.when(b == B - 1)
    def _():
        cols = []
        for g in range(G):
            pg = pool_scr[:, g, :]
            h = jnp.maximum(
                jnp.dot(pg, w1[g], preferred_element_type=jnp.float32)
                + b1[g][None, :], 0.0)
            lgt = jnp.dot(h, w2[g], preferred_element_type=jnp.float32) \
                + b2[g][None, :]
            cols.append(lgt)
        logits = jnp.concatenate(cols, axis=1)
        logits_ref[...] = logits
        logits_t_ref[...] = logits.T
        m = jnp.max(logits, axis=1, keepdims=True)
        e = jnp.exp(logits - m)
        probs = e / jnp.sum(e, axis=1, keepdims=True)
        probs_ref[...] = probs
        imp = jnp.mean(probs, axis=0)
        loss_ref[...] = jnp.full((1, 1), LB_COEF * G) * jnp.sum(imp * imp)


def _pool_mlp(groups2d, w1, b1, w2, b2, interpret=False):
    f32 = jnp.float32
    return pl.pallas_call(
        _pool_mlp_body,
        grid=(B,),
        in_specs=[pl.BlockSpec((RPB, HW), lambda b: (b, 0))
                  for _ in range(G)]
        + [
            pl.BlockSpec((G, C, HIDDEN), lambda b: (0, 0, 0)),
            pl.BlockSpec((G, HIDDEN), lambda b: (0, 0)),
            pl.BlockSpec((G, HIDDEN, 1), lambda b: (0, 0, 0)),
            pl.BlockSpec((G, 1), lambda b: (0, 0)),
        ],
        out_specs=[
            pl.BlockSpec((B, G), lambda b: (0, 0)),
            pl.BlockSpec((G, B), lambda b: (0, 0)),
            pl.BlockSpec((B, G), lambda b: (0, 0)),
            pl.BlockSpec((1, 1), lambda b: (0, 0)),
        ],
        out_shape=[
            jax.ShapeDtypeStruct((B, G), f32),
            jax.ShapeDtypeStruct((G, B), f32),
            jax.ShapeDtypeStruct((B, G), f32),
            jax.ShapeDtypeStruct((1, 1), f32),
        ],
        scratch_shapes=[pltpu.VMEM((B, G, C), f32)],
        interpret=interpret,
    )(*groups2d, w1, b1, w2, b2)


def _route_gather_body(lg_t_hbm, g0, g1, g2, g3, g4, g5, g6, g7,
                       mask_t_hbm, out_hbm,
                       lg_v, mk_v, buf_a, buf_b,
                       sem_la, sem_lb, sem_sa, sem_sb):
    grefs = (g0, g1, g2, g3, g4, g5, g6, g7)
    wid = lax.axis_index("s") * NC + lax.axis_index("c")

    pltpu.sync_copy(lg_t_hbm, lg_v)

    neg = jnp.full((16,), -3.0e38, jnp.float32)
    m1 = neg
    i1 = jnp.zeros((16,), jnp.int32)
    for g in range(G):
        v = lg_v[g]
        better = v > m1
        m1 = jnp.where(better, v, m1)
        i1 = jnp.where(better, g, i1)
    m2 = neg
    i2 = jnp.zeros((16,), jnp.int32)
    for g in range(G):
        v = lg_v[g]
        ok = (v > m2) & (i1 != g)
        m2 = jnp.where(ok, v, m2)
        i2 = jnp.where(ok, g, i2)

    for g in range(G):
        sel = (i1 == g) | (i2 == g)
        mk_v[g] = jnp.where(sel, 1.0, 0.0).astype(jnp.float32)

    @pl.when(wid == 0)
    def _():
        pltpu.sync_copy(mk_v, mask_t_hbm)

    b = wid // K
    k = wid % K
    lane = lax.broadcasted_iota(jnp.int32, (16,), 0)
    sel_ivec = jnp.where(k == 0, i1, i2)

    for g in range(G):
        hitg = (sel_ivec == g) & (lane == b)
        cnt = plsc.all_reduce_population_count(hitg)

        @pl.when(cnt[0] > 0)
        def _(g=g):
            src = grefs[g]
            bufs = (buf_a, buf_b)
            lsems = (sem_la, sem_lb)
            ssems = (sem_sa, sem_sb)

            def ld(c, p):
                return pltpu.async_copy(
                    src.at[pl.ds(b * RPB + c * RCH, RCH)], bufs[p], lsems[p])

            def st(c, p):
                return pltpu.async_copy(
                    bufs[p],
                    out_hbm.at[pl.ds(b * K * RPB + (k * C + c * CCH) * RPC,
                                     RCH)],
                    ssems[p])

            ld(0, 0).wait()
            st_h = [None, None]
            for c in range(NCHUNK):
                p = c % 2
                st_h[p] = st(c, p)
                if c + 1 < NCHUNK:
                    q = (c + 1) % 2
                    if st_h[q] is not None:
                        st_h[q].wait()
                    ld(c + 1, q).wait()
            st_h[0].wait()
            st_h[1].wait()


def _route_gather(logits_t, groups2d):
    f32 = jnp.float32
    run = pl.kernel(
        _route_gather_body,
        out_type=[
            jax.ShapeDtypeStruct((G, B), f32),
            jax.ShapeDtypeStruct((B * K * RPB, HW), f32),
        ],
        mesh=plsc.VectorSubcoreMesh(core_axis_name="c", subcore_axis_name="s"),
        compiler_params=pltpu.CompilerParams(needs_layout_passes=False,
                                             use_tc_tiling_on_sc=True),
        scratch_types=[
            pltpu.VMEM((G, 16), f32),
            pltpu.VMEM((G, 16), f32),
            pltpu.VMEM((RCH, HW), f32),
            pltpu.VMEM((RCH, HW), f32),
            pltpu.SemaphoreType.DMA,
            pltpu.SemaphoreType.DMA,
            pltpu.SemaphoreType.DMA,
            pltpu.SemaphoreType.DMA,
        ],
    )
    return run(logits_t, *groups2d)


def kernel(groups_0, groups_1, groups_2, groups_3, groups_4, groups_5,
           groups_6, groups_7, W1, b1, W2, b2):
    gs = (groups_0, groups_1, groups_2, groups_3, groups_4, groups_5,
          groups_6, groups_7)
    gs2 = tuple(g.reshape(B * C * HW, HW) for g in gs)
    logits, logits_t, soft_probs, loss11 = _pool_mlp(gs2, W1, b1, W2, b2)
    mask_t, out2 = _route_gather(logits_t, gs2)
    out = out2.reshape(B, K * C, HW, HW)
    hard_mask = mask_t.T
    load_loss = loss11[0, 0]
    return (out, logits, hard_mask, soft_probs, load_loss)
